# Initial kernel scaffold; baseline (speedup 1.0000x reference)
import jax
import jax.numpy as jnp
from jax.experimental import pallas as pl


def kernel(x, edge_index, W, b):
    raise NotImplementedError("write your pallas kernel here")


# speedup vs baseline: 4.8987x; 4.8987x over previous
import functools

import jax
import jax.numpy as jnp
from jax import lax
from jax.experimental import pallas as pl
from jax.experimental.pallas import tpu as pltpu
from jax.experimental.pallas import tpu_sc as plsc

N_NODES = 10000
N_EDGES = 320000
IN_CH = 128
OUT_CH = 128

NC = 2
NS = 16
EDGES_PER_TILE = N_EDGES // (NC * NS)
EDGES_PER_S = N_EDGES // NS
CHUNK = 80
N_CHUNKS = EDGES_PER_TILE // CHUNK
N_PAD = 10240
ROWS_PER_TILE = N_PAD // NS
N_GROUPS = ROWS_PER_TILE // 16


def _mm_body(x_ref, wt_ref, b_ref, o_ref):
    o_ref[...] = (
        jnp.dot(x_ref[...], wt_ref[...], preferred_element_type=jnp.float32)
        + b_ref[...]
    )


def _linear(x, wt, b2):
    blk = 2000
    return pl.pallas_call(
        _mm_body,
        grid=(N_NODES // blk,),
        in_specs=[
            pl.BlockSpec((blk, IN_CH), lambda i: (i, 0)),
            pl.BlockSpec((IN_CH, OUT_CH), lambda i: (0, 0)),
            pl.BlockSpec((1, OUT_CH), lambda i: (0, 0)),
        ],
        out_specs=pl.BlockSpec((blk, OUT_CH), lambda i: (i, 0)),
        out_shape=jax.ShapeDtypeStruct((N_NODES, OUT_CH), jnp.float32),
    )(x, wt, b2)


def _sc_body(h_hbm, src_hbm, dst_hbm, zeros_hbm, parts_hbm,
             src_idx, dst_own, dst_oth, rows, deg_v, redbuf, divbuf,
             agg, grid_sp, sem):
    c = lax.axis_index("c")
    s = lax.axis_index("s")

    pltpu.sync_copy(zeros_hbm, agg.at[pl.ds(s * ROWS_PER_TILE, ROWS_PER_TILE)])

    def zero_deg(i, carry):
        deg_v[pl.ds(i * 16, 16)] = jnp.zeros((16,), jnp.float32)
        return carry
    lax.fori_loop(0, N_PAD // 16, zero_deg, 0)

    plsc.subcore_barrier()

    base_own = s * EDGES_PER_S + c * EDGES_PER_TILE
    base_oth = s * EDGES_PER_S + (1 - c) * EDGES_PER_TILE
    ones16 = jnp.ones((16,), jnp.float32)

    def body(j, carry):
        off = j * CHUNK
        pltpu.sync_copy(src_hbm.at[pl.ds(base_own + off, CHUNK)], src_idx)
        pltpu.sync_copy(dst_hbm.at[pl.ds(base_own + off, CHUNK)], dst_own)
        pltpu.sync_copy(dst_hbm.at[pl.ds(base_oth + off, CHUNK)], dst_oth)
        pltpu.async_copy(h_hbm.at[src_idx], rows, sem).wait()
        pltpu.sync_copy(rows, agg.at[dst_own], add=True)
        for k in range(CHUNK // 16):
            plsc.addupdate_scatter(deg_v, [dst_own[pl.ds(k * 16, 16)]], ones16)
            plsc.addupdate_scatter(deg_v, [dst_oth[pl.ds(k * 16, 16)]], ones16)
        return carry

    lax.fori_loop(0, N_CHUNKS, body, 0)

    pltpu.sync_copy(deg_v, grid_sp.at[s])
    plsc.subcore_barrier()
    pltpu.sync_copy(grid_sp.at[:, pl.ds(s * ROWS_PER_TILE, ROWS_PER_TILE)],
                    redbuf)

    iota16 = lax.iota(jnp.int32, 16)
    rowbase = s * ROWS_PER_TILE

    def div_group(g, carry):
        acc = redbuf[0, pl.ds(g * 16, 16)]
        for r in range(1, NS):
            acc = acc + redbuf[r, pl.ds(g * 16, 16)]
        inv = 1.0 / jnp.maximum(acc, 1.0)
        pltpu.sync_copy(agg.at[pl.ds(rowbase + g * 16, 16)], divbuf)
        for j in range(16):
            vj = jnp.sum(jnp.where(iota16 == j, inv, 0.0))
            for k in range(OUT_CH // 16):
                divbuf[j, pl.ds(k * 16, 16)] = divbuf[j, pl.ds(k * 16, 16)] * vj
        pltpu.sync_copy(divbuf, parts_hbm.at[c, pl.ds(rowbase + g * 16, 16)])
        return carry

    lax.fori_loop(0, N_GROUPS, div_group, 0)


_sc_aggregate = functools.partial(
    pl.kernel,
    out_type=jax.ShapeDtypeStruct((NC, N_PAD, OUT_CH), jnp.float32),
    mesh=plsc.VectorSubcoreMesh(core_axis_name="c", subcore_axis_name="s"),
    scratch_types=[
        pltpu.VMEM((CHUNK,), jnp.int32),
        pltpu.VMEM((CHUNK,), jnp.int32),
        pltpu.VMEM((CHUNK,), jnp.int32),
        pltpu.VMEM((CHUNK, OUT_CH), jnp.float32),
        pltpu.VMEM((N_PAD,), jnp.float32),
        pltpu.VMEM((NS, ROWS_PER_TILE), jnp.float32),
        pltpu.VMEM((16, OUT_CH), jnp.float32),
        pltpu.VMEM_SHARED((N_PAD, OUT_CH), jnp.float32),
        pltpu.VMEM_SHARED((NS, N_PAD), jnp.float32),
        pltpu.SemaphoreType.DMA,
    ],
    compiler_params=pltpu.CompilerParams(needs_layout_passes=False),
)(_sc_body)


def _fin_body(p_ref, o_ref):
    o_ref[...] = p_ref[0] + p_ref[1]


def _finalize(parts):
    blk = 2000
    return pl.pallas_call(
        _fin_body,
        grid=(N_NODES // blk,),
        in_specs=[pl.BlockSpec((NC, blk, OUT_CH), lambda i: (0, i, 0))],
        out_specs=pl.BlockSpec((blk, OUT_CH), lambda i: (i, 0)),
        out_shape=jax.ShapeDtypeStruct((N_NODES, OUT_CH), jnp.float32),
    )(parts)


def kernel(x, edge_index, W, b):
    x = x.astype(jnp.float32)
    W = W.astype(jnp.float32)
    b = b.astype(jnp.float32)
    h = _linear(x, W.T, b.reshape(1, OUT_CH))

    dst = edge_index[0].astype(jnp.int32)
    src = edge_index[1].astype(jnp.int32)
    zeros_rows = jnp.zeros((ROWS_PER_TILE, OUT_CH), jnp.float32)
    parts = _sc_aggregate(h, src, dst, zeros_rows)
    return _finalize(parts)


# speedup vs baseline: 7.8471x; 1.6019x over previous
import functools

import jax
import jax.numpy as jnp
from jax import lax
from jax.experimental import pallas as pl
from jax.experimental.pallas import tpu as pltpu
from jax.experimental.pallas import tpu_sc as plsc

N_NODES = 10000
N_EDGES = 320000
IN_CH = 128
OUT_CH = 128

NC = 2
NS = 16
EDGES_PER_TILE = N_EDGES // (NC * NS)
EDGES_PER_S = N_EDGES // NS
CHUNK = 80
N_CHUNKS = EDGES_PER_TILE // CHUNK
N_PAD = 10240
ROWS_PER_TILE = N_PAD // NS
N_GROUPS = ROWS_PER_TILE // 16


def _mm_body(x_ref, wt_ref, b_ref, o_ref):
    o_ref[...] = (
        jnp.dot(x_ref[...], wt_ref[...], preferred_element_type=jnp.float32)
        + b_ref[...]
    )


def _linear(x, wt, b2):
    blk = 2000
    return pl.pallas_call(
        _mm_body,
        grid=(N_NODES // blk,),
        in_specs=[
            pl.BlockSpec((blk, IN_CH), lambda i: (i, 0)),
            pl.BlockSpec((IN_CH, OUT_CH), lambda i: (0, 0)),
            pl.BlockSpec((1, OUT_CH), lambda i: (0, 0)),
        ],
        out_specs=pl.BlockSpec((blk, OUT_CH), lambda i: (i, 0)),
        out_shape=jax.ShapeDtypeStruct((N_NODES, OUT_CH), jnp.float32),
    )(x, wt, b2)


RED_SEC = 128
N_SEC = ROWS_PER_TILE // RED_SEC
GROUPS_PER_SEC = RED_SEC // 16


def _sc_body(h_hbm, src_hbm, dst_hbm, zeros_hbm, parts_hbm,
             src_idx, dst_own, dst_oth, rows, deg_v, redbuf, divbuf,
             agg, grid_sp, sem_i0, sem_i1, sem_g0, sem_g1, sem_s0, sem_s1):
    c = lax.axis_index("c")
    s = lax.axis_index("s")
    sem_i = (sem_i0, sem_i1)
    sem_g = (sem_g0, sem_g1)
    sem_s = (sem_s0, sem_s1)

    pltpu.sync_copy(zeros_hbm, agg.at[pl.ds(s * ROWS_PER_TILE, ROWS_PER_TILE)])

    def zero_deg(i, carry):
        deg_v[pl.ds(i * 16, 16)] = jnp.zeros((16,), jnp.float32)
        return carry
    lax.fori_loop(0, N_PAD // 16, zero_deg, 0)

    plsc.subcore_barrier()

    base_own = s * EDGES_PER_S + c * EDGES_PER_TILE
    base_oth = s * EDGES_PER_S + (1 - c) * EDGES_PER_TILE
    ones16 = jnp.ones((16,), jnp.float32)

    def load_idx(j, b):
        off = j * CHUNK
        pltpu.async_copy(src_hbm.at[pl.ds(base_own + off, CHUNK)],
                         src_idx.at[b], sem_i[b])
        pltpu.async_copy(dst_hbm.at[pl.ds(base_own + off, CHUNK)],
                         dst_own.at[b], sem_i[b])
        pltpu.async_copy(dst_hbm.at[pl.ds(base_oth + off, CHUNK)],
                         dst_oth.at[b], sem_i[b])

    def wait_idx(b):
        pltpu.make_async_copy(src_hbm.at[pl.ds(0, CHUNK)], src_idx.at[b],
                              sem_i[b]).wait()
        pltpu.make_async_copy(dst_hbm.at[pl.ds(0, CHUNK)], dst_own.at[b],
                              sem_i[b]).wait()
        pltpu.make_async_copy(dst_hbm.at[pl.ds(0, CHUNK)], dst_oth.at[b],
                              sem_i[b]).wait()

    def wait_scatter(b):
        pltpu.make_async_copy(rows.at[b], agg.at[dst_own.at[b]],
                              sem_s[b]).wait()

    def deg_update(b):
        for k in range(CHUNK // 16):
            plsc.addupdate_scatter(deg_v, [dst_own[b, pl.ds(k * 16, 16)]],
                                   ones16)
            plsc.addupdate_scatter(deg_v, [dst_oth[b, pl.ds(k * 16, 16)]],
                                   ones16)

    def run_chunk(b):
        wait_idx(b)
        pltpu.async_copy(h_hbm.at[src_idx.at[b]], rows.at[b], sem_g[b])
        deg_update(b)
        pltpu.make_async_copy(h_hbm.at[src_idx.at[b]], rows.at[b],
                              sem_g[b]).wait()
        pltpu.async_copy(rows.at[b], agg.at[dst_own.at[b]], sem_s[b],
                         add=True)

    load_idx(0, 0)

    def pair(g, carry):
        @pl.when(g > 0)
        def _():
            wait_scatter(1)
        load_idx(2 * g + 1, 1)
        run_chunk(0)
        wait_scatter(0)
        load_idx(2 * g + 2, 0)
        run_chunk(1)
        return carry

    lax.fori_loop(0, (N_CHUNKS - 1) // 2, pair, 0)
    wait_scatter(1)
    run_chunk(0)
    wait_scatter(0)

    pltpu.sync_copy(deg_v, grid_sp.at[s])
    plsc.subcore_barrier()

    iota16 = lax.iota(jnp.int32, 16)
    rowbase = s * ROWS_PER_TILE

    def div_section(t, carry):
        pltpu.sync_copy(
            grid_sp.at[:, pl.ds(rowbase + t * RED_SEC, RED_SEC)], redbuf)

        def div_group(g, carry2):
            acc = redbuf[0, pl.ds(g * 16, 16)]
            for r in range(1, NS):
                acc = acc + redbuf[r, pl.ds(g * 16, 16)]
            inv = 1.0 / jnp.maximum(acc, 1.0)
            row0 = rowbase + t * RED_SEC + g * 16
            pltpu.sync_copy(agg.at[pl.ds(row0, 16)], divbuf)
            for j in range(16):
                vj = jnp.sum(jnp.where(iota16 == j, inv, 0.0))
                for k in range(OUT_CH // 16):
                    divbuf[j, pl.ds(k * 16, 16)] = (
                        divbuf[j, pl.ds(k * 16, 16)] * vj)
            pltpu.sync_copy(divbuf, parts_hbm.at[c, pl.ds(row0, 16)])
            return carry2

        lax.fori_loop(0, GROUPS_PER_SEC, div_group, 0)
        return carry

    lax.fori_loop(0, N_SEC, div_section, 0)


_sc_aggregate = functools.partial(
    pl.kernel,
    out_type=jax.ShapeDtypeStruct((NC, N_PAD, OUT_CH), jnp.float32),
    mesh=plsc.VectorSubcoreMesh(core_axis_name="c", subcore_axis_name="s"),
    scratch_types=[
        pltpu.VMEM((2, CHUNK), jnp.int32),
        pltpu.VMEM((2, CHUNK), jnp.int32),
        pltpu.VMEM((2, CHUNK), jnp.int32),
        pltpu.VMEM((2, CHUNK, OUT_CH), jnp.float32),
        pltpu.VMEM((N_PAD,), jnp.float32),
        pltpu.VMEM((NS, RED_SEC), jnp.float32),
        pltpu.VMEM((16, OUT_CH), jnp.float32),
        pltpu.VMEM_SHARED((N_PAD, OUT_CH), jnp.float32),
        pltpu.VMEM_SHARED((NS, N_PAD), jnp.float32),
        pltpu.SemaphoreType.DMA,
        pltpu.SemaphoreType.DMA,
        pltpu.SemaphoreType.DMA,
        pltpu.SemaphoreType.DMA,
        pltpu.SemaphoreType.DMA,
        pltpu.SemaphoreType.DMA,
    ],
    compiler_params=pltpu.CompilerParams(needs_layout_passes=False),
)(_sc_body)


def _fin_body(p_ref, o_ref):
    o_ref[...] = p_ref[0] + p_ref[1]


def _finalize(parts):
    blk = 2000
    return pl.pallas_call(
        _fin_body,
        grid=(N_NODES // blk,),
        in_specs=[pl.BlockSpec((NC, blk, OUT_CH), lambda i: (0, i, 0))],
        out_specs=pl.BlockSpec((blk, OUT_CH), lambda i: (i, 0)),
        out_shape=jax.ShapeDtypeStruct((N_NODES, OUT_CH), jnp.float32),
    )(parts)


def kernel(x, edge_index, W, b):
    x = x.astype(jnp.float32)
    W = W.astype(jnp.float32)
    b = b.astype(jnp.float32)
    h = _linear(x, W.T, b.reshape(1, OUT_CH))

    dst = edge_index[0].astype(jnp.int32)
    src = edge_index[1].astype(jnp.int32)
    zeros_rows = jnp.zeros((ROWS_PER_TILE, OUT_CH), jnp.float32)
    parts = _sc_aggregate(h, src, dst, zeros_rows)
    return _finalize(parts)


# speedup vs baseline: 12.6221x; 1.6085x over previous
import functools

import jax
import jax.numpy as jnp
from jax import lax
from jax.experimental import pallas as pl
from jax.experimental.pallas import tpu as pltpu
from jax.experimental.pallas import tpu_sc as plsc

N_NODES = 10000
N_EDGES = 320000
IN_CH = 128
OUT_CH = 128

NC = 2
NS = 16
EDGES_PER_TILE = N_EDGES // (NC * NS)
EDGES_PER_S = N_EDGES // NS
CHUNK = 80
N_CHUNKS = EDGES_PER_TILE // CHUNK
N_PAD = 10240
ROWS_PER_TILE = N_PAD // NS
NRB = 3
NIB = 4
BODY = 12
N_BODY = 120 // BODY
RED_SEC = 128
N_SEC = ROWS_PER_TILE // RED_SEC
GROUPS_PER_SEC = RED_SEC // 16


def _mm_body(x_ref, wt_ref, b_ref, o_ref):
    o_ref[...] = (
        jnp.dot(x_ref[...], wt_ref[...], preferred_element_type=jnp.float32)
        + b_ref[...]
    )


def _linear(x, wt, b2):
    blk = 2000
    return pl.pallas_call(
        _mm_body,
        grid=(N_NODES // blk,),
        in_specs=[
            pl.BlockSpec((blk, IN_CH), lambda i: (i, 0)),
            pl.BlockSpec((IN_CH, OUT_CH), lambda i: (0, 0)),
            pl.BlockSpec((1, OUT_CH), lambda i: (0, 0)),
        ],
        out_specs=pl.BlockSpec((blk, OUT_CH), lambda i: (i, 0)),
        out_shape=jax.ShapeDtypeStruct((N_NODES, OUT_CH), jnp.float32),
    )(x, wt, b2)


def _sc_body(h_hbm, src_hbm, dst_hbm, zeros_hbm, parts_hbm, degall_hbm,
             src_idx, dst_sc, dst_oth, rows, deg_v, redbuf, divbuf,
             agg,
             sg0, sg1, sg2, ss0, ss1, ss2, si0, si1, si2, si3):
    c = lax.axis_index("c")
    s = lax.axis_index("s")
    sem_g = (sg0, sg1, sg2)
    sem_s = (ss0, ss1, ss2)
    sem_i = (si0, si1, si2, si3)

    pltpu.sync_copy(zeros_hbm, agg.at[pl.ds(s * ROWS_PER_TILE, ROWS_PER_TILE)])

    def zero_deg(i, carry):
        deg_v[pl.ds(i * 16, 16)] = jnp.zeros((16,), jnp.float32)
        return carry
    lax.fori_loop(0, N_PAD // 16, zero_deg, 0)

    plsc.subcore_barrier()

    base_own = s * EDGES_PER_S + c * EDGES_PER_TILE
    base_oth = s * EDGES_PER_S + (1 - c) * EDGES_PER_TILE
    ones16 = jnp.ones((16,), jnp.float32)

    def load_idx(off_chunks, b4):
        off = off_chunks * CHUNK
        pltpu.async_copy(src_hbm.at[pl.ds(base_own + off, CHUNK)],
                         src_idx.at[b4], sem_i[b4])
        pltpu.async_copy(dst_hbm.at[pl.ds(base_own + off, CHUNK)],
                         dst_sc.at[b4], sem_i[b4])
        pltpu.async_copy(dst_hbm.at[pl.ds(base_oth + off, CHUNK)],
                         dst_oth.at[b4], sem_i[b4])

    def wait_idx(b4):
        for dst in (src_idx.at[b4], dst_sc.at[b4], dst_oth.at[b4]):
            pltpu.make_async_copy(src_hbm.at[pl.ds(0, CHUNK)], dst,
                                  sem_i[b4]).wait()

    def stage_a(b3, b4, first_wrap):
        def ws():
            pltpu.make_async_copy(rows.at[b3], agg.at[dst_sc.at[4 + b3]],
                                  sem_s[b3]).wait()
        if first_wrap is None:
            ws()
        else:
            @pl.when(first_wrap)
            def _():
                ws()
        wait_idx(b4)
        pltpu.async_copy(h_hbm.at[src_idx.at[b4]], rows.at[b3], sem_g[b3])

    def stage_b(b3, b4):
        for k in range(CHUNK // 16):
            sl = pl.ds(k * 16, 16)
            plsc.addupdate_scatter(deg_v, [dst_sc[b4, sl]], ones16)
            plsc.addupdate_scatter(deg_v, [dst_oth[b4, sl]], ones16)
            dst_sc[4 + b3, sl] = dst_sc[b4, sl]

    def stage_c(b3, prefetch_chunk, b4):
        pltpu.make_async_copy(h_hbm.at[src_idx.at[b4]], rows.at[b3],
                              sem_g[b3]).wait()
        pltpu.async_copy(rows.at[b3], agg.at[dst_sc.at[4 + b3]], sem_s[b3],
                         add=True)
        if prefetch_chunk is not None:
            load_idx(prefetch_chunk, b4)

    for b4 in range(NIB):
        load_idx(b4, b4)

    def body(g, carry):
        m0 = BODY * g
        for r in range(BODY):
            b3, b4 = r % NRB, r % NIB
            first_wrap = (g > 0) if r < NRB else None
            stage_a(b3, b4, first_wrap)
            stage_b(b3, b4)
            pb3, pb4 = (r - 1) % NRB, (r - 1) % NIB
            if r == 0:
                @pl.when(g > 0)
                def _():
                    stage_c(pb3, None, pb4)
                    load_idx(m0 - 1 + NIB, pb4)
            else:
                stage_c(pb3, None, pb4)
                load_idx(m0 + r - 1 + NIB, pb4)
        return carry

    lax.fori_loop(0, N_BODY, body, 0)

    for m in range(120, N_CHUNKS):
        b3, b4 = m % NRB, m % NIB
        stage_a(b3, b4, None)
        stage_b(b3, b4)
        pb3, pb4 = (m - 1) % NRB, (m - 1) % NIB
        if m - 1 + NIB <= N_CHUNKS - 1:
            stage_c(pb3, m - 1 + NIB, pb4)
        else:
            stage_c(pb3, None, pb4)
    last = N_CHUNKS - 1
    stage_c(last % NRB, None, last % NIB)
    for m in range(N_CHUNKS - NRB, N_CHUNKS):
        b3 = m % NRB
        pltpu.make_async_copy(rows.at[b3], agg.at[dst_sc.at[4 + b3]],
                              sem_s[b3]).wait()

    pltpu.sync_copy(deg_v, degall_hbm.at[c, s])
    plsc.subcore_barrier()

    iota16 = lax.iota(jnp.int32, 16)
    rowbase = s * ROWS_PER_TILE

    def div_section(t, carry):
        pltpu.sync_copy(
            degall_hbm.at[c, :, pl.ds(rowbase + t * RED_SEC, RED_SEC)],
            redbuf)

        def div_group(g, carry2):
            acc = redbuf[0, pl.ds(g * 16, 16)]
            for r in range(1, NS):
                acc = acc + redbuf[r, pl.ds(g * 16, 16)]
            inv = 1.0 / jnp.maximum(acc, 1.0)
            row0 = rowbase + t * RED_SEC + g * 16
            pltpu.sync_copy(agg.at[pl.ds(row0, 16)], divbuf)
            for j in range(16):
                vj = jnp.sum(jnp.where(iota16 == j, inv, 0.0))
                for k in range(OUT_CH // 16):
                    divbuf[j, pl.ds(k * 16, 16)] = (
                        divbuf[j, pl.ds(k * 16, 16)] * vj)
            pltpu.sync_copy(divbuf, parts_hbm.at[c, pl.ds(row0, 16)])
            return carry2

        lax.fori_loop(0, GROUPS_PER_SEC, div_group, 0)
        return carry

    lax.fori_loop(0, N_SEC, div_section, 0)


_sc_aggregate = functools.partial(
    pl.kernel,
    out_type=(
        jax.ShapeDtypeStruct((NC, N_PAD, OUT_CH), jnp.float32),
        jax.ShapeDtypeStruct((NC, NS, N_PAD), jnp.float32),
    ),
    mesh=plsc.VectorSubcoreMesh(core_axis_name="c", subcore_axis_name="s"),
    scratch_types=[
        pltpu.VMEM((NIB, CHUNK), jnp.int32),
        pltpu.VMEM((NIB + NRB, CHUNK), jnp.int32),
        pltpu.VMEM((NIB, CHUNK), jnp.int32),
        pltpu.VMEM((NRB, CHUNK, OUT_CH), jnp.float32),
        pltpu.VMEM((N_PAD,), jnp.float32),
        pltpu.VMEM((NS, RED_SEC), jnp.float32),
        pltpu.VMEM((16, OUT_CH), jnp.float32),
        pltpu.VMEM_SHARED((N_PAD, OUT_CH), jnp.float32),
        pltpu.SemaphoreType.DMA,
        pltpu.SemaphoreType.DMA,
        pltpu.SemaphoreType.DMA,
        pltpu.SemaphoreType.DMA,
        pltpu.SemaphoreType.DMA,
        pltpu.SemaphoreType.DMA,
        pltpu.SemaphoreType.DMA,
        pltpu.SemaphoreType.DMA,
        pltpu.SemaphoreType.DMA,
        pltpu.SemaphoreType.DMA,
    ],
    compiler_params=pltpu.CompilerParams(needs_layout_passes=False),
)(_sc_body)


def _fin_body(p_ref, o_ref):
    o_ref[...] = p_ref[0] + p_ref[1]


def _finalize(parts):
    blk = 2000
    return pl.pallas_call(
        _fin_body,
        grid=(N_NODES // blk,),
        in_specs=[pl.BlockSpec((NC, blk, OUT_CH), lambda i: (0, i, 0))],
        out_specs=pl.BlockSpec((blk, OUT_CH), lambda i: (i, 0)),
        out_shape=jax.ShapeDtypeStruct((N_NODES, OUT_CH), jnp.float32),
    )(parts)


def kernel(x, edge_index, W, b):
    x = x.astype(jnp.float32)
    W = W.astype(jnp.float32)
    b = b.astype(jnp.float32)
    h = _linear(x, W.T, b.reshape(1, OUT_CH))

    dst = edge_index[0].astype(jnp.int32)
    src = edge_index[1].astype(jnp.int32)
    zeros_rows = jnp.zeros((ROWS_PER_TILE, OUT_CH), jnp.float32)
    parts, _ = _sc_aggregate(h, src, dst, zeros_rows)
    return _finalize(parts)


# speedup vs baseline: 13.2013x; 1.0459x over previous
import functools

import jax
import jax.numpy as jnp
from jax import lax
from jax.experimental import pallas as pl
from jax.experimental.pallas import tpu as pltpu
from jax.experimental.pallas import tpu_sc as plsc

N_NODES = 10000
N_EDGES = 320000
IN_CH = 128
OUT_CH = 128

NC = 2
NS = 16
EDGES_PER_TILE = N_EDGES // (NC * NS)
EDGES_PER_S = N_EDGES // NS
CHUNK = 80
N_CHUNKS = EDGES_PER_TILE // CHUNK
N_PAD = 10240
ROWS_PER_TILE = N_PAD // NS
NRB = 3
NIB = 4
BODY = 12
N_BODY = 120 // BODY
RED_SEC = 128
N_SEC = ROWS_PER_TILE // RED_SEC
GROUPS_PER_SEC = RED_SEC // 16


def _mm_body(x_ref, wt_ref, b_ref, o_ref):
    o_ref[...] = (
        jnp.dot(x_ref[...], wt_ref[...], preferred_element_type=jnp.float32)
        + b_ref[...]
    )


def _linear(x, wt, b2):
    blk = 2000
    return pl.pallas_call(
        _mm_body,
        grid=(N_NODES // blk,),
        in_specs=[
            pl.BlockSpec((blk, IN_CH), lambda i: (i, 0)),
            pl.BlockSpec((IN_CH, OUT_CH), lambda i: (0, 0)),
            pl.BlockSpec((1, OUT_CH), lambda i: (0, 0)),
        ],
        out_specs=pl.BlockSpec((blk, OUT_CH), lambda i: (i, 0)),
        out_shape=jax.ShapeDtypeStruct((N_NODES, OUT_CH), jnp.float32),
    )(x, wt, b2)


def _sc_body(h_hbm, src_hbm, dst_hbm, zeros_hbm, parts_hbm, degall_hbm,
             src_idx, dst_sc, dst_oth, rows, deg_v, redbuf,
             agg,
             sg0, sg1, sg2, ss0, ss1, ss2, si0, si1, si2, si3):
    c = lax.axis_index("c")
    s = lax.axis_index("s")
    sem_g = (sg0, sg1, sg2)
    sem_s = (ss0, ss1, ss2)
    sem_i = (si0, si1, si2, si3)

    pltpu.sync_copy(zeros_hbm, agg.at[pl.ds(s * ROWS_PER_TILE, ROWS_PER_TILE)])

    def zero_deg(i, carry):
        deg_v[pl.ds(i * 16, 16)] = jnp.zeros((16,), jnp.float32)
        return carry
    lax.fori_loop(0, N_PAD // 16, zero_deg, 0)

    plsc.subcore_barrier()

    base_own = s * EDGES_PER_S + c * EDGES_PER_TILE
    base_oth = s * EDGES_PER_S + (1 - c) * EDGES_PER_TILE
    ones16 = jnp.ones((16,), jnp.float32)

    def load_idx(off_chunks, b4):
        off = off_chunks * CHUNK
        pltpu.async_copy(src_hbm.at[pl.ds(base_own + off, CHUNK)],
                         src_idx.at[b4], sem_i[b4])
        pltpu.async_copy(dst_hbm.at[pl.ds(base_own + off, CHUNK)],
                         dst_sc.at[b4], sem_i[b4])
        pltpu.async_copy(dst_hbm.at[pl.ds(base_oth + off, CHUNK)],
                         dst_oth.at[b4], sem_i[b4])

    def wait_idx(b4):
        for dst in (src_idx.at[b4], dst_sc.at[b4], dst_oth.at[b4]):
            pltpu.make_async_copy(src_hbm.at[pl.ds(0, CHUNK)], dst,
                                  sem_i[b4]).wait()

    def stage_a(b3, b4, first_wrap):
        def ws():
            pltpu.make_async_copy(rows.at[b3], agg.at[dst_sc.at[4 + b3]],
                                  sem_s[b3]).wait()
        if first_wrap is None:
            ws()
        else:
            @pl.when(first_wrap)
            def _():
                ws()
        wait_idx(b4)
        pltpu.async_copy(h_hbm.at[src_idx.at[b4]], rows.at[b3], sem_g[b3])

    def stage_b(b3, b4):
        for k in range(CHUNK // 16):
            sl = pl.ds(k * 16, 16)
            plsc.addupdate_scatter(deg_v, [dst_sc[b4, sl]], ones16)
            plsc.addupdate_scatter(deg_v, [dst_oth[b4, sl]], ones16)
            dst_sc[4 + b3, sl] = dst_sc[b4, sl]

    def stage_c(b3, prefetch_chunk, b4):
        pltpu.make_async_copy(h_hbm.at[src_idx.at[b4]], rows.at[b3],
                              sem_g[b3]).wait()
        pltpu.async_copy(rows.at[b3], agg.at[dst_sc.at[4 + b3]], sem_s[b3],
                         add=True)
        if prefetch_chunk is not None:
            load_idx(prefetch_chunk, b4)

    for b4 in range(NIB):
        load_idx(b4, b4)

    def body(g, carry):
        m0 = BODY * g
        for r in range(BODY):
            b3, b4 = r % NRB, r % NIB
            first_wrap = (g > 0) if r < NRB else None
            stage_a(b3, b4, first_wrap)
            stage_b(b3, b4)
            pb3, pb4 = (r - 1) % NRB, (r - 1) % NIB
            if r == 0:
                @pl.when(g > 0)
                def _():
                    stage_c(pb3, None, pb4)
                    load_idx(m0 - 1 + NIB, pb4)
            else:
                stage_c(pb3, None, pb4)
                load_idx(m0 + r - 1 + NIB, pb4)
        return carry

    lax.fori_loop(0, N_BODY, body, 0)

    for m in range(120, N_CHUNKS):
        b3, b4 = m % NRB, m % NIB
        stage_a(b3, b4, None)
        stage_b(b3, b4)
        pb3, pb4 = (m - 1) % NRB, (m - 1) % NIB
        if m - 1 + NIB <= N_CHUNKS - 1:
            stage_c(pb3, m - 1 + NIB, pb4)
        else:
            stage_c(pb3, None, pb4)
    last = N_CHUNKS - 1
    stage_c(last % NRB, None, last % NIB)
    for m in range(N_CHUNKS - NRB, N_CHUNKS):
        b3 = m % NRB
        pltpu.make_async_copy(rows.at[b3], agg.at[dst_sc.at[4 + b3]],
                              sem_s[b3]).wait()

    pltpu.sync_copy(deg_v, degall_hbm.at[c, s])
    plsc.subcore_barrier()

    rowbase = s * ROWS_PER_TILE
    n_groups = ROWS_PER_TILE // 16
    slots = (rows.at[0, pl.ds(0, 16)], rows.at[0, pl.ds(16, 16)])
    sem_in = (sem_g[0], sem_g[1])
    sem_out = (sem_s[0], sem_s[1])

    def start_in(p, sl):
        pltpu.async_copy(agg.at[pl.ds(rowbase + p * 16, 16)], slots[sl],
                         sem_in[sl])

    def wait_in(sl):
        pltpu.make_async_copy(agg.at[pl.ds(0, 16)], slots[sl],
                              sem_in[sl]).wait()

    def start_out(p, sl):
        pltpu.async_copy(slots[sl], parts_hbm.at[c, pl.ds(rowbase + p * 16, 16)],
                         sem_out[sl])

    def wait_out(sl):
        pltpu.make_async_copy(slots[sl], parts_hbm.at[c, pl.ds(0, 16)],
                              sem_out[sl]).wait()

    iota16 = lax.iota(jnp.int32, 16)

    def compute(p, sl):
        off = (p % GROUPS_PER_SEC) * 16
        acc = redbuf[0, pl.ds(off, 16)]
        for r in range(1, NS):
            acc = acc + redbuf[r, pl.ds(off, 16)]
        inv = 1.0 / jnp.maximum(acc, 1.0)
        buf = slots[sl]
        for j in range(16):
            vj = jnp.sum(jnp.where(iota16 == j, inv, 0.0))
            for k in range(OUT_CH // 16):
                buf[j, pl.ds(k * 16, 16)] = buf[j, pl.ds(k * 16, 16)] * vj

    start_in(0, 0)

    def div_pair(q, carry):
        p0 = 2 * q
        @pl.when(q > 0)
        def _():
            wait_out(1)
        start_in(p0 + 1, 1)

        @pl.when(lax.rem(q, 4) == 0)
        def _():
            pltpu.sync_copy(
                degall_hbm.at[c, :, pl.ds(rowbase + (q // 4) * RED_SEC,
                                          RED_SEC)],
                redbuf)

        wait_in(0)
        compute(p0, 0)
        start_out(p0, 0)

        wait_in(1)
        compute(p0 + 1, 1)
        start_out(p0 + 1, 1)

        @pl.when(q < n_groups // 2 - 1)
        def _():
            wait_out(0)
            start_in(p0 + 2, 0)
        return carry

    lax.fori_loop(0, n_groups // 2, div_pair, 0)
    wait_out(0)
    wait_out(1)


_sc_aggregate = functools.partial(
    pl.kernel,
    out_type=(
        jax.ShapeDtypeStruct((NC, N_PAD, OUT_CH), jnp.float32),
        jax.ShapeDtypeStruct((NC, NS, N_PAD), jnp.float32),
    ),
    mesh=plsc.VectorSubcoreMesh(core_axis_name="c", subcore_axis_name="s"),
    scratch_types=[
        pltpu.VMEM((NIB, CHUNK), jnp.int32),
        pltpu.VMEM((NIB + NRB, CHUNK), jnp.int32),
        pltpu.VMEM((NIB, CHUNK), jnp.int32),
        pltpu.VMEM((NRB, CHUNK, OUT_CH), jnp.float32),
        pltpu.VMEM((N_PAD,), jnp.float32),
        pltpu.VMEM((NS, RED_SEC), jnp.float32),
        pltpu.VMEM_SHARED((N_PAD, OUT_CH), jnp.float32),
        pltpu.SemaphoreType.DMA,
        pltpu.SemaphoreType.DMA,
        pltpu.SemaphoreType.DMA,
        pltpu.SemaphoreType.DMA,
        pltpu.SemaphoreType.DMA,
        pltpu.SemaphoreType.DMA,
        pltpu.SemaphoreType.DMA,
        pltpu.SemaphoreType.DMA,
        pltpu.SemaphoreType.DMA,
        pltpu.SemaphoreType.DMA,
    ],
    compiler_params=pltpu.CompilerParams(needs_layout_passes=False),
)(_sc_body)


def _fin_body(p_ref, o_ref):
    o_ref[...] = p_ref[0] + p_ref[1]


def _finalize(parts):
    blk = 2000
    return pl.pallas_call(
        _fin_body,
        grid=(N_NODES // blk,),
        in_specs=[pl.BlockSpec((NC, blk, OUT_CH), lambda i: (0, i, 0))],
        out_specs=pl.BlockSpec((blk, OUT_CH), lambda i: (i, 0)),
        out_shape=jax.ShapeDtypeStruct((N_NODES, OUT_CH), jnp.float32),
    )(parts)


def kernel(x, edge_index, W, b):
    x = x.astype(jnp.float32)
    W = W.astype(jnp.float32)
    b = b.astype(jnp.float32)
    h = _linear(x, W.T, b.reshape(1, OUT_CH))

    dst = edge_index[0].astype(jnp.int32)
    src = edge_index[1].astype(jnp.int32)
    zeros_rows = jnp.zeros((ROWS_PER_TILE, OUT_CH), jnp.float32)
    parts, _ = _sc_aggregate(h, src, dst, zeros_rows)
    return _finalize(parts)


# speedup vs baseline: 13.4765x; 1.0208x over previous
import functools

import jax
import jax.numpy as jnp
from jax import lax
from jax.experimental import pallas as pl
from jax.experimental.pallas import tpu as pltpu
from jax.experimental.pallas import tpu_sc as plsc

N_NODES = 10000
N_EDGES = 320000
IN_CH = 128
OUT_CH = 128

NC = 2
NS = 16
EDGES_PER_TILE = N_EDGES // (NC * NS)
EDGES_PER_S = N_EDGES // NS
CHUNK = 80
N_CHUNKS = EDGES_PER_TILE // CHUNK
N_PAD = 10240
ROWS_PER_TILE = N_PAD // NS
NRB = 3
NIB = 4
BODY = 12
N_BODY = 120 // BODY
RED_SEC = 128
N_SEC = ROWS_PER_TILE // RED_SEC
GROUPS_PER_SEC = RED_SEC // 16


def _mm_body(x_ref, wt_ref, b_ref, o_ref):
    o_ref[...] = (
        jnp.dot(x_ref[...], wt_ref[...], preferred_element_type=jnp.float32)
        + b_ref[...]
    )


def _linear(x, wt, b2):
    blk = 2000
    return pl.pallas_call(
        _mm_body,
        grid=(N_NODES // blk,),
        in_specs=[
            pl.BlockSpec((blk, IN_CH), lambda i: (i, 0)),
            pl.BlockSpec((IN_CH, OUT_CH), lambda i: (0, 0)),
            pl.BlockSpec((1, OUT_CH), lambda i: (0, 0)),
        ],
        out_specs=pl.BlockSpec((blk, OUT_CH), lambda i: (i, 0)),
        out_shape=jax.ShapeDtypeStruct((N_NODES, OUT_CH), jnp.float32),
    )(x, wt, b2)


def _sc_body(h_hbm, src_hbm, dst_hbm, zeros_hbm, parts_hbm, degall_hbm,
             src_idx, dst_sc, dst_oth, rows, deg_v, redbuf,
             agg,
             sg0, sg1, sg2, ss0, ss1, ss2, si0, si1, si2, si3):
    c = lax.axis_index("c")
    s = lax.axis_index("s")
    sem_g = (sg0, sg1, sg2)
    sem_s = (ss0, ss1, ss2)
    sem_i = (si0, si1, si2, si3)

    base_own = s * EDGES_PER_S + c * EDGES_PER_TILE
    base_oth = s * EDGES_PER_S + (1 - c) * EDGES_PER_TILE
    ones16 = jnp.ones((16,), jnp.float32)

    def load_idx(off_chunks, b4):
        off = off_chunks * CHUNK
        pltpu.async_copy(src_hbm.at[pl.ds(base_own + off, CHUNK)],
                         src_idx.at[b4], sem_i[b4])
        pltpu.async_copy(dst_hbm.at[pl.ds(base_own + off, CHUNK)],
                         dst_sc.at[b4], sem_i[b4])
        pltpu.async_copy(dst_hbm.at[pl.ds(base_oth + off, CHUNK)],
                         dst_oth.at[b4], sem_i[b4])

    def wait_idx(b4):
        for dst in (src_idx.at[b4], dst_sc.at[b4], dst_oth.at[b4]):
            pltpu.make_async_copy(src_hbm.at[pl.ds(0, CHUNK)], dst,
                                  sem_i[b4]).wait()

    def stage_a(b3, b4, first_wrap):
        def ws():
            pltpu.make_async_copy(rows.at[b3], agg.at[dst_sc.at[4 + b3]],
                                  sem_s[b3]).wait()
        if first_wrap is None:
            ws()
        else:
            @pl.when(first_wrap)
            def _():
                ws()
        wait_idx(b4)
        pltpu.async_copy(h_hbm.at[src_idx.at[b4]], rows.at[b3], sem_g[b3])

    def stage_b(b3, b4):
        for k in range(CHUNK // 16):
            sl = pl.ds(k * 16, 16)
            plsc.addupdate_scatter(deg_v, [dst_sc[b4, sl]], ones16)
            plsc.addupdate_scatter(deg_v, [dst_oth[b4, sl]], ones16)
            dst_sc[4 + b3, sl] = dst_sc[b4, sl]

    def stage_c(b3, prefetch_chunk, b4):
        pltpu.make_async_copy(h_hbm.at[src_idx.at[b4]], rows.at[b3],
                              sem_g[b3]).wait()
        pltpu.async_copy(rows.at[b3], agg.at[dst_sc.at[4 + b3]], sem_s[b3],
                         add=True)
        if prefetch_chunk is not None:
            load_idx(prefetch_chunk, b4)

    for b4 in range(NIB):
        load_idx(b4, b4)

    pltpu.async_copy(zeros_hbm,
                     agg.at[pl.ds(s * ROWS_PER_TILE, ROWS_PER_TILE)],
                     sem_g[0])

    def zero_deg(i, carry):
        deg_v[pl.ds(i * 16, 16)] = jnp.zeros((16,), jnp.float32)
        return carry
    lax.fori_loop(0, N_PAD // 16, zero_deg, 0)

    pltpu.make_async_copy(
        zeros_hbm, agg.at[pl.ds(s * ROWS_PER_TILE, ROWS_PER_TILE)],
        sem_g[0]).wait()
    plsc.subcore_barrier()

    def body(g, carry):
        m0 = BODY * g
        for r in range(BODY):
            b3, b4 = r % NRB, r % NIB
            first_wrap = (g > 0) if r < NRB else None
            stage_a(b3, b4, first_wrap)
            stage_b(b3, b4)
            pb3, pb4 = (r - 1) % NRB, (r - 1) % NIB
            if r == 0:
                @pl.when(g > 0)
                def _():
                    stage_c(pb3, None, pb4)
                    load_idx(m0 - 1 + NIB, pb4)
            else:
                stage_c(pb3, None, pb4)
                load_idx(m0 + r - 1 + NIB, pb4)
        return carry

    lax.fori_loop(0, N_BODY, body, 0)

    for m in range(120, N_CHUNKS):
        b3, b4 = m % NRB, m % NIB
        stage_a(b3, b4, None)
        stage_b(b3, b4)
        pb3, pb4 = (m - 1) % NRB, (m - 1) % NIB
        if m - 1 + NIB <= N_CHUNKS - 1:
            stage_c(pb3, m - 1 + NIB, pb4)
        else:
            stage_c(pb3, None, pb4)
    last = N_CHUNKS - 1
    stage_c(last % NRB, None, last % NIB)
    for m in range(N_CHUNKS - NRB, N_CHUNKS):
        b3 = m % NRB
        pltpu.make_async_copy(rows.at[b3], agg.at[dst_sc.at[4 + b3]],
                              sem_s[b3]).wait()

    pltpu.sync_copy(deg_v, degall_hbm.at[c, s])
    plsc.subcore_barrier()

    rowbase = s * ROWS_PER_TILE
    n_groups = ROWS_PER_TILE // 16
    slots = (rows.at[0, pl.ds(0, 16)], rows.at[0, pl.ds(16, 16)])
    sem_in = (sem_g[0], sem_g[1])
    sem_out = (sem_s[0], sem_s[1])

    def start_in(p, sl):
        pltpu.async_copy(agg.at[pl.ds(rowbase + p * 16, 16)], slots[sl],
                         sem_in[sl])

    def wait_in(sl):
        pltpu.make_async_copy(agg.at[pl.ds(0, 16)], slots[sl],
                              sem_in[sl]).wait()

    def start_out(p, sl):
        pltpu.async_copy(slots[sl], parts_hbm.at[c, pl.ds(rowbase + p * 16, 16)],
                         sem_out[sl])

    def wait_out(sl):
        pltpu.make_async_copy(slots[sl], parts_hbm.at[c, pl.ds(0, 16)],
                              sem_out[sl]).wait()

    iota16 = lax.iota(jnp.int32, 16)

    def compute(p, sl):
        off = (p % GROUPS_PER_SEC) * 16
        acc = redbuf[0, pl.ds(off, 16)]
        for r in range(1, NS):
            acc = acc + redbuf[r, pl.ds(off, 16)]
        inv = 1.0 / jnp.maximum(acc, 1.0)
        buf = slots[sl]
        for j in range(16):
            vj = jnp.sum(jnp.where(iota16 == j, inv, 0.0))
            for k in range(OUT_CH // 16):
                buf[j, pl.ds(k * 16, 16)] = buf[j, pl.ds(k * 16, 16)] * vj

    start_in(0, 0)

    def div_pair(q, carry):
        p0 = 2 * q
        @pl.when(q > 0)
        def _():
            wait_out(1)
        start_in(p0 + 1, 1)

        @pl.when(lax.rem(q, 4) == 0)
        def _():
            pltpu.sync_copy(
                degall_hbm.at[c, :, pl.ds(rowbase + (q // 4) * RED_SEC,
                                          RED_SEC)],
                redbuf)

        wait_in(0)
        compute(p0, 0)
        start_out(p0, 0)

        wait_in(1)
        compute(p0 + 1, 1)
        start_out(p0 + 1, 1)

        @pl.when(q < n_groups // 2 - 1)
        def _():
            wait_out(0)
            start_in(p0 + 2, 0)
        return carry

    lax.fori_loop(0, n_groups // 2, div_pair, 0)
    wait_out(0)
    wait_out(1)


_sc_aggregate = functools.partial(
    pl.kernel,
    out_type=(
        jax.ShapeDtypeStruct((NC, N_PAD, OUT_CH), jnp.float32),
        jax.ShapeDtypeStruct((NC, NS, N_PAD), jnp.float32),
    ),
    mesh=plsc.VectorSubcoreMesh(core_axis_name="c", subcore_axis_name="s"),
    scratch_types=[
        pltpu.VMEM((NIB, CHUNK), jnp.int32),
        pltpu.VMEM((NIB + NRB, CHUNK), jnp.int32),
        pltpu.VMEM((NIB, CHUNK), jnp.int32),
        pltpu.VMEM((NRB, CHUNK, OUT_CH), jnp.float32),
        pltpu.VMEM((N_PAD,), jnp.float32),
        pltpu.VMEM((NS, RED_SEC), jnp.float32),
        pltpu.VMEM_SHARED((N_PAD, OUT_CH), jnp.float32),
        pltpu.SemaphoreType.DMA,
        pltpu.SemaphoreType.DMA,
        pltpu.SemaphoreType.DMA,
        pltpu.SemaphoreType.DMA,
        pltpu.SemaphoreType.DMA,
        pltpu.SemaphoreType.DMA,
        pltpu.SemaphoreType.DMA,
        pltpu.SemaphoreType.DMA,
        pltpu.SemaphoreType.DMA,
        pltpu.SemaphoreType.DMA,
    ],
    compiler_params=pltpu.CompilerParams(needs_layout_passes=False),
)(_sc_body)


def _fin_body(p_ref, o_ref):
    o_ref[...] = p_ref[0] + p_ref[1]


def _finalize(parts):
    blk = 2000
    return pl.pallas_call(
        _fin_body,
        grid=(N_NODES // blk,),
        in_specs=[pl.BlockSpec((NC, blk, OUT_CH), lambda i: (0, i, 0))],
        out_specs=pl.BlockSpec((blk, OUT_CH), lambda i: (i, 0)),
        out_shape=jax.ShapeDtypeStruct((N_NODES, OUT_CH), jnp.float32),
    )(parts)


def kernel(x, edge_index, W, b):
    x = x.astype(jnp.float32)
    W = W.astype(jnp.float32)
    b = b.astype(jnp.float32)
    h = _linear(x, W.T, b.reshape(1, OUT_CH))

    dst = edge_index[0].astype(jnp.int32)
    src = edge_index[1].astype(jnp.int32)
    zeros_rows = jnp.zeros((ROWS_PER_TILE, OUT_CH), jnp.float32)
    parts, _ = _sc_aggregate(h, src, dst, zeros_rows)
    return _finalize(parts)
